# baseline (device time: 33468 ns/iter reference)
import jax
import jax.numpy as jnp
from jax import lax
from jax.experimental import pallas as pl
from jax.experimental.pallas import tpu as pltpu

B, S, D = 2, 256, 1024
M = B * S
DC_SH = 64
DC = 2 * DC_SH
H, DH, DR = 16, 64, 32
HL = H // 2
QW = HL * DH
QRW = HL * DR
NC = 2
HC = HL // NC
CW = HC * DH
SCALE = (DH + DR) ** -0.5

CP_X, CP_DKV, CP_Q, CP_QR, CP_KR, CP_WO, CP_UK, CP_UV = range(8)
RD_UK, RD_UV, RD_C = range(3)


def kernel(x, Wdkv, Wuk, Wuv, Wq, Wqr, Wkr, Wo):
    def body(x_ref, wdkv_ref, wuk_ref, wuv_ref, wq_ref, wqr_ref, wkr_ref,
             wo_ref, out_ref,
             xv, wdkv_v, wq_v, wqr_v, wkr_v, wo_v,
             cm, cp, ccat, wukcat, wuvcat, q, qr, kr, k, v, oc, op, scb,
             copy_sems, send_sems, recv_sems):
        my_x = lax.axis_index("x")
        my_y = lax.axis_index("y")
        peer_y = (my_x, 1 - my_y)
        peer_x = (1 - my_x, my_y)
        hcol = my_x * QW
        qrcol = my_x * QRW

        copies = {
            CP_X: pltpu.make_async_copy(x_ref, xv, copy_sems.at[CP_X]),
            CP_DKV: pltpu.make_async_copy(wdkv_ref, wdkv_v,
                                          copy_sems.at[CP_DKV]),
            CP_Q: pltpu.make_async_copy(wq_ref.at[:, pl.ds(hcol, QW)],
                                        wq_v, copy_sems.at[CP_Q]),
            CP_QR: pltpu.make_async_copy(wqr_ref.at[:, pl.ds(qrcol, QRW)],
                                         wqr_v, copy_sems.at[CP_QR]),
            CP_KR: pltpu.make_async_copy(wkr_ref, wkr_v, copy_sems.at[CP_KR]),
            CP_UK: pltpu.make_async_copy(wuk_ref.at[:, pl.ds(hcol, QW)],
                                         wukcat.at[pl.ds(0, DC_SH), :],
                                         copy_sems.at[CP_UK]),
            CP_UV: pltpu.make_async_copy(wuv_ref.at[:, pl.ds(hcol, QW)],
                                         wuvcat.at[pl.ds(0, DC_SH), :],
                                         copy_sems.at[CP_UV]),
            CP_WO: pltpu.make_async_copy(wo_ref, wo_v, copy_sems.at[CP_WO]),
        }
        for i in (CP_X, CP_DKV, CP_Q, CP_QR, CP_KR):
            copies[i].start()

        barrier = pltpu.get_barrier_semaphore()
        for nbr in (peer_y, peer_x):
            pl.semaphore_signal(barrier, inc=1, device_id=nbr,
                                device_id_type=pl.DeviceIdType.MESH)
        pl.semaphore_wait(barrier, 2)

        wuk_rdma = pltpu.make_async_remote_copy(
            src_ref=wuk_ref.at[:, pl.ds(hcol, QW)],
            dst_ref=wukcat.at[pl.ds(DC_SH, DC_SH), :],
            send_sem=send_sems.at[RD_UK], recv_sem=recv_sems.at[RD_UK],
            device_id=peer_y, device_id_type=pl.DeviceIdType.MESH)
        wuk_rdma.start()
        wuv_rdma = pltpu.make_async_remote_copy(
            src_ref=wuv_ref.at[:, pl.ds(hcol, QW)],
            dst_ref=wuvcat.at[pl.ds(DC_SH, DC_SH), :],
            send_sem=send_sems.at[RD_UV], recv_sem=recv_sems.at[RD_UV],
            device_id=peer_y, device_id_type=pl.DeviceIdType.MESH)
        wuv_rdma.start()

        copies[CP_X].wait()
        copies[CP_DKV].wait()
        x2 = jnp.reshape(xv[...], (M, D))
        c_mine = lax.dot_general(x2, wdkv_v[...], (((1,), (1,)), ((), ())),
                                 preferred_element_type=jnp.float32)
        cm[...] = c_mine
        ccat[:, 0:DC_SH] = c_mine
        c_rdma = pltpu.make_async_remote_copy(
            src_ref=cm, dst_ref=cp,
            send_sem=send_sems.at[RD_C], recv_sem=recv_sems.at[RD_C],
            device_id=peer_y, device_id_type=pl.DeviceIdType.MESH)
        c_rdma.start()

        for i in (CP_UK, CP_UV, CP_WO):
            copies[i].start()

        copies[CP_Q].wait()
        q[...] = jnp.dot(x2, wq_v[...],
                         preferred_element_type=jnp.float32) * SCALE
        copies[CP_QR].wait()
        qr[...] = jnp.dot(x2, wqr_v[...],
                          preferred_element_type=jnp.float32) * SCALE
        copies[CP_KR].wait()
        kr[...] = lax.dot_general(x2, wkr_v[...], (((1,), (1,)), ((), ())),
                                  preferred_element_type=jnp.float32)

        copies[CP_UK].wait()
        copies[CP_UV].wait()
        wuk_rdma.wait()
        wuv_rdma.wait()
        c_rdma.wait()
        ccat[:, DC_SH:DC] = cp[...]

        k[...] = jnp.dot(ccat[...], wukcat[...],
                         preferred_element_type=jnp.float32)
        v[...] = jnp.dot(ccat[...], wuvcat[...],
                         preferred_element_type=jnp.float32)

        o_rdmas = []
        for ci in range(NC):
            for b in range(B):
                r = slice(b * S, (b + 1) * S)
                krb = kr[r, :]
                for hl in range(HC):
                    idx = b * HC + hl
                    h = ci * HC + hl
                    qh = q[r, h * DH:(h + 1) * DH]
                    kh = k[r, h * DH:(h + 1) * DH]
                    qrh = qr[r, h * DR:(h + 1) * DR]
                    scb[idx * S:(idx + 1) * S, :] = (
                        lax.dot_general(qh, kh, (((1,), (1,)), ((), ())),
                                        preferred_element_type=jnp.float32)
                        + lax.dot_general(qrh, krb,
                                          (((1,), (1,)), ((), ())),
                                          preferred_element_type=jnp.float32))
            p = jnp.exp(scb[...])
            inv = 1.0 / jnp.sum(p, axis=1, keepdims=True)
            for b in range(B):
                r = slice(b * S, (b + 1) * S)
                for hl in range(HC):
                    idx = b * HC + hl
                    h = ci * HC + hl
                    vh = v[r, h * DH:(h + 1) * DH]
                    ps = p[idx * S:(idx + 1) * S, :]
                    o_raw = jnp.dot(ps, vh, preferred_element_type=jnp.float32)
                    oc[ci, r, hl * DH:(hl + 1) * DH] = (
                        o_raw * inv[idx * S:(idx + 1) * S, :]
                    ).astype(jnp.bfloat16)
            rdma = pltpu.make_async_remote_copy(
                src_ref=oc.at[ci], dst_ref=op.at[ci],
                send_sem=send_sems.at[3 + ci], recv_sem=recv_sems.at[3 + ci],
                device_id=peer_x, device_id_type=pl.DeviceIdType.MESH)
            rdma.start()
            o_rdmas.append(rdma)

        copies[CP_WO].wait()
        wo_row = my_x * QW
        out2 = jnp.dot(oc[0].astype(jnp.float32),
                       wo_v[pl.ds(wo_row, CW), :],
                       preferred_element_type=jnp.float32)
        for ci in range(1, NC):
            out2 = out2 + jnp.dot(oc[ci].astype(jnp.float32),
                                  wo_v[pl.ds(wo_row + ci * CW, CW), :],
                                  preferred_element_type=jnp.float32)
        wo_row_p = (1 - my_x) * QW
        for ci in range(NC):
            o_rdmas[ci].wait()
            out2 = out2 + jnp.dot(op[ci].astype(jnp.float32),
                                  wo_v[pl.ds(wo_row_p + ci * CW, CW), :],
                                  preferred_element_type=jnp.float32)

        out_ref[...] = jnp.reshape(out2, (B, S, D))

    return pl.pallas_call(
        body,
        out_shape=jax.ShapeDtypeStruct((B, S, D), jnp.float32),
        in_specs=[pl.BlockSpec(memory_space=pltpu.MemorySpace.HBM)] * 8,
        out_specs=pl.BlockSpec(memory_space=pltpu.VMEM),
        scratch_shapes=[
            pltpu.VMEM((B, S, D), jnp.float32),
            pltpu.VMEM((DC_SH, D), jnp.float32),
            pltpu.VMEM((D, QW), jnp.float32),
            pltpu.VMEM((D, QRW), jnp.float32),
            pltpu.VMEM((DR, D), jnp.float32),
            pltpu.VMEM((D, D), jnp.float32),
            pltpu.VMEM((M, DC_SH), jnp.float32),
            pltpu.VMEM((M, DC_SH), jnp.float32),
            pltpu.VMEM((M, DC), jnp.float32),
            pltpu.VMEM((DC, QW), jnp.float32),
            pltpu.VMEM((DC, QW), jnp.float32),
            pltpu.VMEM((M, QW), jnp.float32),
            pltpu.VMEM((M, QRW), jnp.float32),
            pltpu.VMEM((M, DR), jnp.float32),
            pltpu.VMEM((M, QW), jnp.float32),
            pltpu.VMEM((M, QW), jnp.float32),
            pltpu.VMEM((NC, M, CW), jnp.bfloat16),
            pltpu.VMEM((NC, M, CW), jnp.bfloat16),
            pltpu.VMEM((B * HC * S, S), jnp.float32),
            pltpu.SemaphoreType.DMA((8,)),
            pltpu.SemaphoreType.DMA((7,)),
            pltpu.SemaphoreType.DMA((7,)),
        ],
        compiler_params=pltpu.CompilerParams(collective_id=0),
    )(x, Wdkv.T, Wuk, Wuv, Wq, Wqr, Wkr.T, Wo)


# device time: 32933 ns/iter; 1.0162x vs baseline; 1.0162x over previous
import jax
import jax.numpy as jnp
from jax import lax
from jax.experimental import pallas as pl
from jax.experimental.pallas import tpu as pltpu

B, S, D = 2, 256, 1024
M = B * S
DC_SH = 64
DC = 2 * DC_SH
H, DH, DR = 16, 64, 32
HL = H // 2
QW = HL * DH
QRW = HL * DR
NC = 4
HC = HL // NC
CW = HC * DH
SCALE = (DH + DR) ** -0.5

CP_X, CP_DKV, CP_Q, CP_QR, CP_KR, CP_WO, CP_UK, CP_UV = range(8)
RD_UK, RD_UV, RD_C = range(3)


def kernel(x, Wdkv, Wuk, Wuv, Wq, Wqr, Wkr, Wo):
    def body(x_ref, wdkv_ref, wuk_ref, wuv_ref, wq_ref, wqr_ref, wkr_ref,
             wo_ref, out_ref,
             xv, wdkv_v, wq_v, wqr_v, wkr_v, wo_v,
             cm, cp, ccat, wukcat, wuvcat, q, qr, kr, k, v, oc, op, scb,
             copy_sems, send_sems, recv_sems):
        my_x = lax.axis_index("x")
        my_y = lax.axis_index("y")
        peer_y = (my_x, 1 - my_y)
        peer_x = (1 - my_x, my_y)
        hcol = my_x * QW
        qrcol = my_x * QRW

        copies = {
            CP_X: pltpu.make_async_copy(x_ref, xv, copy_sems.at[CP_X]),
            CP_DKV: pltpu.make_async_copy(wdkv_ref, wdkv_v,
                                          copy_sems.at[CP_DKV]),
            CP_Q: pltpu.make_async_copy(wq_ref.at[:, pl.ds(hcol, QW)],
                                        wq_v, copy_sems.at[CP_Q]),
            CP_QR: pltpu.make_async_copy(wqr_ref.at[:, pl.ds(qrcol, QRW)],
                                         wqr_v, copy_sems.at[CP_QR]),
            CP_KR: pltpu.make_async_copy(wkr_ref, wkr_v, copy_sems.at[CP_KR]),
            CP_UK: pltpu.make_async_copy(wuk_ref.at[:, pl.ds(hcol, QW)],
                                         wukcat.at[pl.ds(0, DC_SH), :],
                                         copy_sems.at[CP_UK]),
            CP_UV: pltpu.make_async_copy(wuv_ref.at[:, pl.ds(hcol, QW)],
                                         wuvcat.at[pl.ds(0, DC_SH), :],
                                         copy_sems.at[CP_UV]),
            CP_WO: pltpu.make_async_copy(wo_ref, wo_v, copy_sems.at[CP_WO]),
        }
        for i in (CP_X, CP_DKV, CP_Q, CP_QR, CP_KR):
            copies[i].start()

        barrier = pltpu.get_barrier_semaphore()
        for nbr in (peer_y, peer_x):
            pl.semaphore_signal(barrier, inc=1, device_id=nbr,
                                device_id_type=pl.DeviceIdType.MESH)
        pl.semaphore_wait(barrier, 2)

        wuk_rdma = pltpu.make_async_remote_copy(
            src_ref=wuk_ref.at[:, pl.ds(hcol, QW)],
            dst_ref=wukcat.at[pl.ds(DC_SH, DC_SH), :],
            send_sem=send_sems.at[RD_UK], recv_sem=recv_sems.at[RD_UK],
            device_id=peer_y, device_id_type=pl.DeviceIdType.MESH)
        wuk_rdma.start()
        wuv_rdma = pltpu.make_async_remote_copy(
            src_ref=wuv_ref.at[:, pl.ds(hcol, QW)],
            dst_ref=wuvcat.at[pl.ds(DC_SH, DC_SH), :],
            send_sem=send_sems.at[RD_UV], recv_sem=recv_sems.at[RD_UV],
            device_id=peer_y, device_id_type=pl.DeviceIdType.MESH)
        wuv_rdma.start()

        copies[CP_X].wait()
        copies[CP_DKV].wait()
        x2 = jnp.reshape(xv[...], (M, D))
        c_mine = lax.dot_general(x2, wdkv_v[...], (((1,), (1,)), ((), ())),
                                 preferred_element_type=jnp.float32)
        cm[...] = c_mine
        ccat[:, 0:DC_SH] = c_mine
        c_rdma = pltpu.make_async_remote_copy(
            src_ref=cm, dst_ref=cp,
            send_sem=send_sems.at[RD_C], recv_sem=recv_sems.at[RD_C],
            device_id=peer_y, device_id_type=pl.DeviceIdType.MESH)
        c_rdma.start()

        for i in (CP_UK, CP_UV, CP_WO):
            copies[i].start()

        copies[CP_Q].wait()
        q[...] = jnp.dot(x2, wq_v[...],
                         preferred_element_type=jnp.float32) * SCALE
        copies[CP_QR].wait()
        qr[...] = jnp.dot(x2, wqr_v[...],
                          preferred_element_type=jnp.float32) * SCALE
        copies[CP_KR].wait()
        kr[...] = lax.dot_general(x2, wkr_v[...], (((1,), (1,)), ((), ())),
                                  preferred_element_type=jnp.float32)

        copies[CP_UK].wait()
        copies[CP_UV].wait()
        wuk_rdma.wait()
        wuv_rdma.wait()
        c_rdma.wait()
        ccat[:, DC_SH:DC] = cp[...]

        k[...] = jnp.dot(ccat[...], wukcat[...],
                         preferred_element_type=jnp.float32)
        v[...] = jnp.dot(ccat[...], wuvcat[...],
                         preferred_element_type=jnp.float32)

        o_rdmas = []
        for ci in range(NC):
            for b in range(B):
                r = slice(b * S, (b + 1) * S)
                krb = kr[r, :]
                for hl in range(HC):
                    idx = b * HC + hl
                    h = ci * HC + hl
                    qh = q[r, h * DH:(h + 1) * DH]
                    kh = k[r, h * DH:(h + 1) * DH]
                    qrh = qr[r, h * DR:(h + 1) * DR]
                    scb[idx * S:(idx + 1) * S, :] = (
                        lax.dot_general(qh, kh, (((1,), (1,)), ((), ())),
                                        preferred_element_type=jnp.float32)
                        + lax.dot_general(qrh, krb,
                                          (((1,), (1,)), ((), ())),
                                          preferred_element_type=jnp.float32))
            p = jnp.exp(scb[...])
            inv = 1.0 / jnp.sum(p, axis=1, keepdims=True)
            for b in range(B):
                r = slice(b * S, (b + 1) * S)
                for hl in range(HC):
                    idx = b * HC + hl
                    h = ci * HC + hl
                    vh = v[r, h * DH:(h + 1) * DH]
                    ps = p[idx * S:(idx + 1) * S, :]
                    o_raw = jnp.dot(ps, vh, preferred_element_type=jnp.float32)
                    oc[ci, r, hl * DH:(hl + 1) * DH] = (
                        o_raw * inv[idx * S:(idx + 1) * S, :]
                    ).astype(jnp.bfloat16)
            rdma = pltpu.make_async_remote_copy(
                src_ref=oc.at[ci], dst_ref=op.at[ci],
                send_sem=send_sems.at[3 + ci], recv_sem=recv_sems.at[3 + ci],
                device_id=peer_x, device_id_type=pl.DeviceIdType.MESH)
            rdma.start()
            o_rdmas.append(rdma)

        copies[CP_WO].wait()
        wo_row = my_x * QW
        out2 = jnp.dot(oc[0].astype(jnp.float32),
                       wo_v[pl.ds(wo_row, CW), :],
                       preferred_element_type=jnp.float32)
        for ci in range(1, NC):
            out2 = out2 + jnp.dot(oc[ci].astype(jnp.float32),
                                  wo_v[pl.ds(wo_row + ci * CW, CW), :],
                                  preferred_element_type=jnp.float32)
        wo_row_p = (1 - my_x) * QW
        for ci in range(NC):
            o_rdmas[ci].wait()
            out2 = out2 + jnp.dot(op[ci].astype(jnp.float32),
                                  wo_v[pl.ds(wo_row_p + ci * CW, CW), :],
                                  preferred_element_type=jnp.float32)

        out_ref[...] = jnp.reshape(out2, (B, S, D))

    return pl.pallas_call(
        body,
        out_shape=jax.ShapeDtypeStruct((B, S, D), jnp.float32),
        in_specs=[pl.BlockSpec(memory_space=pltpu.MemorySpace.HBM)] * 8,
        out_specs=pl.BlockSpec(memory_space=pltpu.VMEM),
        scratch_shapes=[
            pltpu.VMEM((B, S, D), jnp.float32),
            pltpu.VMEM((DC_SH, D), jnp.float32),
            pltpu.VMEM((D, QW), jnp.float32),
            pltpu.VMEM((D, QRW), jnp.float32),
            pltpu.VMEM((DR, D), jnp.float32),
            pltpu.VMEM((D, D), jnp.float32),
            pltpu.VMEM((M, DC_SH), jnp.float32),
            pltpu.VMEM((M, DC_SH), jnp.float32),
            pltpu.VMEM((M, DC), jnp.float32),
            pltpu.VMEM((DC, QW), jnp.float32),
            pltpu.VMEM((DC, QW), jnp.float32),
            pltpu.VMEM((M, QW), jnp.float32),
            pltpu.VMEM((M, QRW), jnp.float32),
            pltpu.VMEM((M, DR), jnp.float32),
            pltpu.VMEM((M, QW), jnp.float32),
            pltpu.VMEM((M, QW), jnp.float32),
            pltpu.VMEM((NC, M, CW), jnp.bfloat16),
            pltpu.VMEM((NC, M, CW), jnp.bfloat16),
            pltpu.VMEM((B * HC * S, S), jnp.float32),
            pltpu.SemaphoreType.DMA((8,)),
            pltpu.SemaphoreType.DMA((7,)),
            pltpu.SemaphoreType.DMA((7,)),
        ],
        compiler_params=pltpu.CompilerParams(collective_id=0),
    )(x, Wdkv.T, Wuk, Wuv, Wq, Wqr, Wkr.T, Wo)
